# baseline (speedup 1.0000x reference)
import functools

import jax
import jax.numpy as jnp
from jax import lax
from jax.experimental import pallas as pl
from jax.experimental.pallas import tpu as pltpu
from jax.experimental.pallas import tpu_sc as plsc

N = 10000
D = 128
E = 320000
NC = 2
NS = 16
CHUNK = 128
CH0 = 80
CH1 = 80
GRP = 40
E_PAD = NS * (CH0 + CH1) * CHUNK
NROWS = 10112
RPS = NROWS // NS

_MESH = plsc.VectorSubcoreMesh(
    core_axis_name="c", subcore_axis_name="s", num_cores=NC, num_subcores=NS
)


def _sc_degree(dst0, dst1, ones_blk, zeros_feat):
    @functools.partial(
        pl.kernel,
        out_type=(jax.ShapeDtypeStruct((NROWS, D), jnp.float32),
                  jax.ShapeDtypeStruct((NROWS, D), jnp.float32)),
        mesh=_MESH,
        scratch_types=[
            pltpu.VMEM((GRP, CHUNK), jnp.int32),
            pltpu.VMEM((CHUNK, D), jnp.float32),
            pltpu.SemaphoreType.DMA,
            pltpu.VMEM_SHARED((NROWS, D), jnp.float32),
        ],
    )
    def k(dst0_hbm, dst1_hbm, ones_hbm, zeros_hbm, out0_hbm, out1_hbm,
          idx_v, ones_v, sem_s, acc):
        cid = lax.axis_index("c")
        sid = lax.axis_index("s")
        pltpu.sync_copy(zeros_hbm.at[pl.ds(sid * RPS, RPS)],
                        acc.at[pl.ds(sid * RPS, RPS)])
        pltpu.sync_copy(ones_hbm, ones_v)
        plsc.subcore_barrier()

        def count(dst_hbm, nch):
            @pl.loop(0, nch // GRP)
            def _(g):
                pltpu.sync_copy(dst_hbm.at[sid, pl.ds(g * GRP, GRP)], idx_v)

                @pl.loop(0, GRP)
                def _(j):
                    pltpu.async_copy(ones_v, acc.at[idx_v.at[j]], sem_s,
                                     add=True)

                @pl.loop(0, GRP)
                def _(j):
                    pltpu.make_async_copy(ones_v, acc.at[idx_v.at[j]],
                                          sem_s).wait()

        @pl.when(cid == 0)
        def _():
            count(dst0_hbm, CH0)

        @pl.when(cid == 1)
        def _():
            count(dst1_hbm, CH1)

        plsc.subcore_barrier()

        @pl.when(cid == 0)
        def _():
            pltpu.sync_copy(acc.at[pl.ds(sid * RPS, RPS)],
                            out0_hbm.at[pl.ds(sid * RPS, RPS)])

        @pl.when(cid == 1)
        def _():
            pltpu.sync_copy(acc.at[pl.ds(sid * RPS, RPS)],
                            out1_hbm.at[pl.ds(sid * RPS, RPS)])

    return k(dst0, dst1, ones_blk, zeros_feat)


def _sc_aggregate(hprime, src0, dst0, src1, dst1, zeros_feat):
    @functools.partial(
        pl.kernel,
        out_type=(jax.ShapeDtypeStruct((NROWS, D), jnp.float32),
                  jax.ShapeDtypeStruct((NROWS, D), jnp.float32)),
        mesh=_MESH,
        scratch_types=[
            pltpu.VMEM((GRP, CHUNK), jnp.int32),
            pltpu.VMEM((GRP, CHUNK), jnp.int32),
            pltpu.VMEM((CHUNK, D), jnp.float32),
            pltpu.VMEM((CHUNK, D), jnp.float32),
            pltpu.SemaphoreType.DMA,
            pltpu.SemaphoreType.DMA,
            pltpu.SemaphoreType.DMA,
            pltpu.SemaphoreType.DMA,
            pltpu.VMEM_SHARED((NROWS, D), jnp.float32),
        ],
    )
    def k(h_hbm, src0_hbm, dst0_hbm, src1_hbm, dst1_hbm, zeros_hbm,
          out0_hbm, out1_hbm, src_v, dst_v, buf_a, buf_b,
          sem_a, sem_b, sem_sa, sem_sb, acc):
        cid = lax.axis_index("c")
        sid = lax.axis_index("s")

        @pl.when(cid == 0)
        def _():
            pltpu.sync_copy(h_hbm.at[pl.ds(sid * RPS, RPS)],
                            acc.at[pl.ds(sid * RPS, RPS)])

        @pl.when(cid != 0)
        def _():
            pltpu.sync_copy(zeros_hbm.at[pl.ds(sid * RPS, RPS)],
                            acc.at[pl.ds(sid * RPS, RPS)])

        plsc.subcore_barrier()

        def aggregate(src_hbm, dst_hbm, nch):
            @pl.loop(0, nch // GRP)
            def _(g):
                pltpu.sync_copy(src_hbm.at[sid, pl.ds(g * GRP, GRP)], src_v)
                pltpu.sync_copy(dst_hbm.at[sid, pl.ds(g * GRP, GRP)], dst_v)
                pltpu.async_copy(h_hbm.at[src_v.at[0]], buf_a, sem_a)
                pltpu.async_copy(h_hbm.at[src_v.at[1]], buf_b, sem_b)

                @pl.loop(0, GRP, step=2)
                def _(j):
                    pltpu.make_async_copy(h_hbm.at[src_v.at[j]], buf_a, sem_a).wait()
                    pltpu.async_copy(buf_a, acc.at[dst_v.at[j]], sem_sa, add=True)
                    pltpu.make_async_copy(h_hbm.at[src_v.at[j + 1]], buf_b, sem_b).wait()
                    pltpu.async_copy(buf_b, acc.at[dst_v.at[j + 1]], sem_sb, add=True)
                    pltpu.make_async_copy(buf_a, acc.at[dst_v.at[j]], sem_sa).wait()

                    @pl.when(j + 2 < GRP)
                    def _():
                        pltpu.async_copy(h_hbm.at[src_v.at[j + 2]], buf_a, sem_a)

                    pltpu.make_async_copy(buf_b, acc.at[dst_v.at[j + 1]], sem_sb).wait()

                    @pl.when(j + 3 < GRP)
                    def _():
                        pltpu.async_copy(h_hbm.at[src_v.at[j + 3]], buf_b, sem_b)

        @pl.when(cid == 0)
        def _():
            aggregate(src0_hbm, dst0_hbm, CH0)

        @pl.when(cid == 1)
        def _():
            aggregate(src1_hbm, dst1_hbm, CH1)

        plsc.subcore_barrier()

        @pl.when(cid == 0)
        def _():
            pltpu.sync_copy(acc.at[pl.ds(sid * RPS, RPS)],
                            out0_hbm.at[pl.ds(sid * RPS, RPS)])

        @pl.when(cid == 1)
        def _():
            pltpu.sync_copy(acc.at[pl.ds(sid * RPS, RPS)],
                            out1_hbm.at[pl.ds(sid * RPS, RPS)])

    return k(hprime, src0, dst0, src1, dst1, zeros_feat)


_BR = 632


def _mm_body(x_ref, w_ref, o_ref):
    o_ref[...] = lax.dot_general(
        x_ref[...], w_ref[...], (((1,), (0,)), ((), ())),
        precision=lax.Precision.HIGHEST, preferred_element_type=jnp.float32)


def _tc_matmul(x, w):
    return pl.pallas_call(
        _mm_body,
        grid=(NROWS // _BR,),
        in_specs=[pl.BlockSpec((_BR, D), lambda i: (i, 0)),
                  pl.BlockSpec((D, D), lambda i: (0, 0))],
        out_specs=pl.BlockSpec((_BR, D), lambda i: (i, 0)),
        out_shape=jax.ShapeDtypeStruct((NROWS, D), jnp.float32),
    )(x, w)


def _dsq(dp0_ref, dp1_ref):
    deg = dp0_ref[:, 0:1] + dp1_ref[:, 0:1] + 1.0
    return lax.rsqrt(deg)


def _scale_body(h_ref, dp0_ref, dp1_ref, o_ref):
    o_ref[...] = h_ref[...] * _dsq(dp0_ref, dp1_ref)


def _tc_scale(h, dp0, dp1):
    return pl.pallas_call(
        _scale_body,
        grid=(NROWS // _BR,),
        in_specs=[pl.BlockSpec((_BR, D), lambda i: (i, 0)),
                  pl.BlockSpec((_BR, D), lambda i: (i, 0)),
                  pl.BlockSpec((_BR, D), lambda i: (i, 0))],
        out_specs=pl.BlockSpec((_BR, D), lambda i: (i, 0)),
        out_shape=jax.ShapeDtypeStruct((NROWS, D), jnp.float32),
    )(h, dp0, dp1)


def _combine1_body(p0_ref, p1_ref, dp0_ref, dp1_ref, b_ref, w_ref, o_ref):
    dsq = _dsq(dp0_ref, dp1_ref)
    h = dsq * (p0_ref[...] + p1_ref[...]) + b_ref[...]
    h = jnp.maximum(h, 0.0)
    h2 = lax.dot_general(h, w_ref[...], (((1,), (0,)), ((), ())),
                         precision=lax.Precision.HIGHEST,
                         preferred_element_type=jnp.float32)
    o_ref[...] = dsq * h2


def _tc_combine1(p0, p1, dp0, dp1, b1, w2):
    return pl.pallas_call(
        _combine1_body,
        grid=(NROWS // _BR,),
        in_specs=[pl.BlockSpec((_BR, D), lambda i: (i, 0)),
                  pl.BlockSpec((_BR, D), lambda i: (i, 0)),
                  pl.BlockSpec((_BR, D), lambda i: (i, 0)),
                  pl.BlockSpec((_BR, D), lambda i: (i, 0)),
                  pl.BlockSpec((1, D), lambda i: (0, 0)),
                  pl.BlockSpec((D, D), lambda i: (0, 0))],
        out_specs=pl.BlockSpec((_BR, D), lambda i: (i, 0)),
        out_shape=jax.ShapeDtypeStruct((NROWS, D), jnp.float32),
    )(p0, p1, dp0, dp1, b1, w2)


def _combine2_body(q0_ref, q1_ref, dp0_ref, dp1_ref, b_ref, o_ref):
    o_ref[...] = (_dsq(dp0_ref, dp1_ref) * (q0_ref[...] + q1_ref[...])
                  + b_ref[...])


def _tc_combine2(q0, q1, dp0, dp1, b2):
    return pl.pallas_call(
        _combine2_body,
        grid=(NROWS // _BR,),
        in_specs=[pl.BlockSpec((_BR, D), lambda i: (i, 0)),
                  pl.BlockSpec((_BR, D), lambda i: (i, 0)),
                  pl.BlockSpec((_BR, D), lambda i: (i, 0)),
                  pl.BlockSpec((_BR, D), lambda i: (i, 0)),
                  pl.BlockSpec((1, D), lambda i: (0, 0))],
        out_specs=pl.BlockSpec((_BR, D), lambda i: (i, 0)),
        out_shape=jax.ShapeDtypeStruct((NROWS, D), jnp.float32),
    )(q0, q1, dp0, dp1, b2)


def kernel(x, edge_index, W1, b1, W2, b2):
    x = jnp.pad(x, ((0, NROWS - N), (0, 0)))
    ei = edge_index.astype(jnp.int32)
    pad = E_PAD - E
    pad_src = jnp.arange(pad, dtype=jnp.int32) % N
    src = jnp.concatenate([ei[0], pad_src])
    dst = jnp.concatenate([ei[1], jnp.full((pad,), N, jnp.int32)])
    n0 = NS * CH0 * CHUNK
    src0 = src[:n0].reshape(NS, CH0, CHUNK)
    dst0 = dst[:n0].reshape(NS, CH0, CHUNK)
    src1 = src[n0:].reshape(NS, CH1, CHUNK)
    dst1 = dst[n0:].reshape(NS, CH1, CHUNK)
    ones_blk = jnp.ones((CHUNK, D), jnp.float32)
    zeros_feat = jnp.zeros((NROWS, D), jnp.float32)

    dp0, dp1 = _sc_degree(dst0, dst1, ones_blk, zeros_feat)
    h1 = _tc_matmul(x, W1)
    h1p = _tc_scale(h1, dp0, dp1)
    p0, p1 = _sc_aggregate(h1p, src0, dst0, src1, dst1, zeros_feat)
    h2p = _tc_combine1(p0, p1, dp0, dp1, b1.reshape(1, D), W2)
    q0, q1 = _sc_aggregate(h2p, src0, dst0, src1, dst1, zeros_feat)
    out = _tc_combine2(q0, q1, dp0, dp1, b2.reshape(1, D))
    return out[:N]


# speedup vs baseline: 1.1907x; 1.1907x over previous
import functools

import jax
import jax.numpy as jnp
from jax import lax
from jax.experimental import pallas as pl
from jax.experimental.pallas import tpu as pltpu
from jax.experimental.pallas import tpu_sc as plsc

N = 10000
D = 128
E = 320000
NC = 2
NS = 16
CHUNK = 128
CH0 = 80
CH1 = 80
GRP = 40
E_PAD = NS * (CH0 + CH1) * CHUNK
NROWS = 10112
RPS = NROWS // NS

_MESH = plsc.VectorSubcoreMesh(
    core_axis_name="c", subcore_axis_name="s", num_cores=NC, num_subcores=NS
)


def _sc_degree(dst0, dst1, ones_blk, zeros_feat):
    @functools.partial(
        pl.kernel,
        out_type=(jax.ShapeDtypeStruct((NROWS, D), jnp.float32),
                  jax.ShapeDtypeStruct((NROWS, D), jnp.float32)),
        mesh=_MESH,
        scratch_types=[
            pltpu.VMEM((GRP, CHUNK), jnp.int32),
            pltpu.VMEM((CHUNK, D), jnp.float32),
            pltpu.SemaphoreType.DMA,
            pltpu.VMEM_SHARED((NROWS, D), jnp.float32),
        ],
    )
    def k(dst0_hbm, dst1_hbm, ones_hbm, zeros_hbm, out0_hbm, out1_hbm,
          idx_v, ones_v, sem_s, acc):
        cid = lax.axis_index("c")
        sid = lax.axis_index("s")
        pltpu.sync_copy(zeros_hbm.at[pl.ds(sid * RPS, RPS)],
                        acc.at[pl.ds(sid * RPS, RPS)])
        pltpu.sync_copy(ones_hbm, ones_v)
        plsc.subcore_barrier()

        def count(dst_hbm, nch):
            @pl.loop(0, nch // GRP)
            def _(g):
                pltpu.sync_copy(dst_hbm.at[sid, pl.ds(g * GRP, GRP)], idx_v)

                @pl.loop(0, GRP)
                def _(j):
                    pltpu.async_copy(ones_v, acc.at[idx_v.at[j]], sem_s,
                                     add=True)

                @pl.loop(0, GRP)
                def _(j):
                    pltpu.make_async_copy(ones_v, acc.at[idx_v.at[j]],
                                          sem_s).wait()

        @pl.when(cid == 0)
        def _():
            count(dst0_hbm, CH0)

        @pl.when(cid == 1)
        def _():
            count(dst1_hbm, CH1)

        plsc.subcore_barrier()

        @pl.when(cid == 0)
        def _():
            pltpu.sync_copy(acc.at[pl.ds(sid * RPS, RPS)],
                            out0_hbm.at[pl.ds(sid * RPS, RPS)])

        @pl.when(cid == 1)
        def _():
            pltpu.sync_copy(acc.at[pl.ds(sid * RPS, RPS)],
                            out1_hbm.at[pl.ds(sid * RPS, RPS)])

    return k(dst0, dst1, ones_blk, zeros_feat)


def _sc_aggregate(hprime, src0, dst0, src1, dst1, zeros_feat):
    @functools.partial(
        pl.kernel,
        out_type=(jax.ShapeDtypeStruct((NROWS, D), jnp.float32),
                  jax.ShapeDtypeStruct((NROWS, D), jnp.float32)),
        mesh=_MESH,
        scratch_types=[
            pltpu.VMEM((GRP, CHUNK), jnp.int32),
            pltpu.VMEM((GRP, CHUNK), jnp.int32),
            pltpu.VMEM((CHUNK, D), jnp.float32),
            pltpu.VMEM((CHUNK, D), jnp.float32),
            pltpu.SemaphoreType.DMA,
            pltpu.SemaphoreType.DMA,
            pltpu.VMEM_SHARED((NROWS, D), jnp.float32),
        ],
    )
    def k(h_hbm, src0_hbm, dst0_hbm, src1_hbm, dst1_hbm, zeros_hbm,
          out0_hbm, out1_hbm, src_v, dst_v, buf_a, buf_b,
          sem_a, sem_b, acc):
        cid = lax.axis_index("c")
        sid = lax.axis_index("s")

        @pl.when(cid == 0)
        def _():
            pltpu.sync_copy(h_hbm.at[pl.ds(sid * RPS, RPS)],
                            acc.at[pl.ds(sid * RPS, RPS)])

        @pl.when(cid != 0)
        def _():
            pltpu.sync_copy(zeros_hbm.at[pl.ds(sid * RPS, RPS)],
                            acc.at[pl.ds(sid * RPS, RPS)])

        plsc.subcore_barrier()

        def aggregate(src_hbm, dst_hbm, nch):
            @pl.loop(0, nch // GRP)
            def _(g):
                pltpu.sync_copy(src_hbm.at[sid, pl.ds(g * GRP, GRP)], src_v)
                pltpu.sync_copy(dst_hbm.at[sid, pl.ds(g * GRP, GRP)], dst_v)
                pltpu.async_copy(h_hbm.at[src_v.at[0]], buf_a, sem_a)

                @pl.loop(0, GRP, step=2)
                def _(j):
                    pltpu.async_copy(h_hbm.at[src_v.at[j + 1]], buf_b, sem_b)
                    pltpu.make_async_copy(h_hbm.at[src_v.at[j]], buf_a, sem_a).wait()
                    pltpu.sync_copy(buf_a, acc.at[dst_v.at[j]], add=True)

                    @pl.when(j + 2 < GRP)
                    def _():
                        pltpu.async_copy(h_hbm.at[src_v.at[j + 2]], buf_a, sem_a)

                    pltpu.make_async_copy(h_hbm.at[src_v.at[j + 1]], buf_b, sem_b).wait()
                    pltpu.sync_copy(buf_b, acc.at[dst_v.at[j + 1]], add=True)

        @pl.when(cid == 0)
        def _():
            aggregate(src0_hbm, dst0_hbm, CH0)

        @pl.when(cid == 1)
        def _():
            aggregate(src1_hbm, dst1_hbm, CH1)

        plsc.subcore_barrier()

        @pl.when(cid == 0)
        def _():
            pltpu.sync_copy(acc.at[pl.ds(sid * RPS, RPS)],
                            out0_hbm.at[pl.ds(sid * RPS, RPS)])

        @pl.when(cid == 1)
        def _():
            pltpu.sync_copy(acc.at[pl.ds(sid * RPS, RPS)],
                            out1_hbm.at[pl.ds(sid * RPS, RPS)])

    return k(hprime, src0, dst0, src1, dst1, zeros_feat)


_BR = 632


def _mm_body(x_ref, w_ref, o_ref):
    o_ref[...] = lax.dot_general(
        x_ref[...], w_ref[...], (((1,), (0,)), ((), ())),
        precision=lax.Precision.HIGHEST, preferred_element_type=jnp.float32)


def _tc_matmul(x, w):
    return pl.pallas_call(
        _mm_body,
        grid=(NROWS // _BR,),
        in_specs=[pl.BlockSpec((_BR, D), lambda i: (i, 0)),
                  pl.BlockSpec((D, D), lambda i: (0, 0))],
        out_specs=pl.BlockSpec((_BR, D), lambda i: (i, 0)),
        out_shape=jax.ShapeDtypeStruct((NROWS, D), jnp.float32),
    )(x, w)


def _dsq(dp0_ref, dp1_ref):
    deg = dp0_ref[:, 0:1] + dp1_ref[:, 0:1] + 1.0
    return lax.rsqrt(deg)


def _scale_body(h_ref, dp0_ref, dp1_ref, o_ref):
    o_ref[...] = h_ref[...] * _dsq(dp0_ref, dp1_ref)


def _tc_scale(h, dp0, dp1):
    return pl.pallas_call(
        _scale_body,
        grid=(NROWS // _BR,),
        in_specs=[pl.BlockSpec((_BR, D), lambda i: (i, 0)),
                  pl.BlockSpec((_BR, D), lambda i: (i, 0)),
                  pl.BlockSpec((_BR, D), lambda i: (i, 0))],
        out_specs=pl.BlockSpec((_BR, D), lambda i: (i, 0)),
        out_shape=jax.ShapeDtypeStruct((NROWS, D), jnp.float32),
    )(h, dp0, dp1)


def _combine1_body(p0_ref, p1_ref, dp0_ref, dp1_ref, b_ref, w_ref, o_ref):
    dsq = _dsq(dp0_ref, dp1_ref)
    h = dsq * (p0_ref[...] + p1_ref[...]) + b_ref[...]
    h = jnp.maximum(h, 0.0)
    h2 = lax.dot_general(h, w_ref[...], (((1,), (0,)), ((), ())),
                         precision=lax.Precision.HIGHEST,
                         preferred_element_type=jnp.float32)
    o_ref[...] = dsq * h2


def _tc_combine1(p0, p1, dp0, dp1, b1, w2):
    return pl.pallas_call(
        _combine1_body,
        grid=(NROWS // _BR,),
        in_specs=[pl.BlockSpec((_BR, D), lambda i: (i, 0)),
                  pl.BlockSpec((_BR, D), lambda i: (i, 0)),
                  pl.BlockSpec((_BR, D), lambda i: (i, 0)),
                  pl.BlockSpec((_BR, D), lambda i: (i, 0)),
                  pl.BlockSpec((1, D), lambda i: (0, 0)),
                  pl.BlockSpec((D, D), lambda i: (0, 0))],
        out_specs=pl.BlockSpec((_BR, D), lambda i: (i, 0)),
        out_shape=jax.ShapeDtypeStruct((NROWS, D), jnp.float32),
    )(p0, p1, dp0, dp1, b1, w2)


def _combine2_body(q0_ref, q1_ref, dp0_ref, dp1_ref, b_ref, o_ref):
    o_ref[...] = (_dsq(dp0_ref, dp1_ref) * (q0_ref[...] + q1_ref[...])
                  + b_ref[...])


def _tc_combine2(q0, q1, dp0, dp1, b2):
    return pl.pallas_call(
        _combine2_body,
        grid=(NROWS // _BR,),
        in_specs=[pl.BlockSpec((_BR, D), lambda i: (i, 0)),
                  pl.BlockSpec((_BR, D), lambda i: (i, 0)),
                  pl.BlockSpec((_BR, D), lambda i: (i, 0)),
                  pl.BlockSpec((_BR, D), lambda i: (i, 0)),
                  pl.BlockSpec((1, D), lambda i: (0, 0))],
        out_specs=pl.BlockSpec((_BR, D), lambda i: (i, 0)),
        out_shape=jax.ShapeDtypeStruct((NROWS, D), jnp.float32),
    )(q0, q1, dp0, dp1, b2)


def kernel(x, edge_index, W1, b1, W2, b2):
    x = jnp.pad(x, ((0, NROWS - N), (0, 0)))
    ei = edge_index.astype(jnp.int32)
    pad = E_PAD - E
    pad_src = jnp.arange(pad, dtype=jnp.int32) % N
    src = jnp.concatenate([ei[0], pad_src])
    dst = jnp.concatenate([ei[1], jnp.full((pad,), N, jnp.int32)])
    n0 = NS * CH0 * CHUNK
    src0 = src[:n0].reshape(NS, CH0, CHUNK)
    dst0 = dst[:n0].reshape(NS, CH0, CHUNK)
    src1 = src[n0:].reshape(NS, CH1, CHUNK)
    dst1 = dst[n0:].reshape(NS, CH1, CHUNK)
    ones_blk = jnp.ones((CHUNK, D), jnp.float32)
    zeros_feat = jnp.zeros((NROWS, D), jnp.float32)

    dp0, dp1 = _sc_degree(dst0, dst1, ones_blk, zeros_feat)
    h1 = _tc_matmul(x, W1)
    h1p = _tc_scale(h1, dp0, dp1)
    p0, p1 = _sc_aggregate(h1p, src0, dst0, src1, dst1, zeros_feat)
    h2p = _tc_combine1(p0, p1, dp0, dp1, b1.reshape(1, D), W2)
    q0, q1 = _sc_aggregate(h2p, src0, dst0, src1, dst1, zeros_feat)
    out = _tc_combine2(q0, q1, dp0, dp1, b2.reshape(1, D))
    return out[:N]
